# Initial kernel scaffold; baseline (speedup 1.0000x reference)
import jax
import jax.numpy as jnp
from jax.experimental import pallas as pl


def kernel(feature, norm, edge_index, W, b):
    raise NotImplementedError("write your pallas kernel here")


# speedup vs baseline: 5.3171x; 5.3171x over previous
import functools

import jax
import jax.numpy as jnp
from jax import lax
from jax.experimental import pallas as pl
from jax.experimental.pallas import tpu as pltpu
from jax.experimental.pallas import tpu_sc as plsc

N_NODES = 10000
D = 128
N_EDGES = 320000
NC, NS, L = 2, 16, 16
NW = NC * NS
E_PER_TILE = N_EDGES // NW
CHUNK = 80
N_CHUNKS = E_PER_TILE // CHUNK
PAD_NODES = 10112
ROWS_PER_TILE = PAD_NODES // NS
ZROWS = 8

_BLK = 1000
_GRID = N_NODES // _BLK


def _h_body(f_ref, n_ref, o_ref):
    o_ref[...] = f_ref[...] * n_ref[...]


def _final_body(p_ref, n_ref, w_ref, b_ref, o_ref):
    agg = (p_ref[0] + p_ref[1]) * n_ref[...]
    o_ref[...] = lax.dot_general(
        agg, w_ref[...], (((1,), (1,)), ((), ())),
        preferred_element_type=jnp.float32) + b_ref[...]


def _sc_body(src_hbm, dst_hbm, h_hbm, out_hbm,
             sidx, didx, rows, zbuf, agg_sh, sem):
    cid = lax.axis_index("c")
    sid = lax.axis_index("s")
    wid = sid * NC + cid
    ebase = wid * E_PER_TILE
    stripe = sid * ROWS_PER_TILE

    for r in range(ZROWS):
        for c in range(D // L):
            zbuf[r, pl.ds(c * L, L)] = jnp.zeros((L,), jnp.float32)
    zsteps = ROWS_PER_TILE // ZROWS

    def zcopy(i, carry):
        pltpu.sync_copy(zbuf, agg_sh.at[pl.ds(stripe + i * ZROWS, ZROWS)])
        return carry

    lax.fori_loop(0, zsteps, zcopy, 0)
    plsc.subcore_barrier()

    def body(j, carry):
        eoff = ebase + j * CHUNK
        pltpu.sync_copy(src_hbm.at[pl.ds(eoff, CHUNK)], sidx)
        pltpu.sync_copy(dst_hbm.at[pl.ds(eoff, CHUNK)], didx)
        pltpu.async_copy(h_hbm.at[sidx], rows, sem).wait()
        pltpu.sync_copy(rows, agg_sh.at[didx], add=True)
        return carry

    lax.fori_loop(0, N_CHUNKS, body, 0)
    plsc.subcore_barrier()

    pltpu.sync_copy(agg_sh.at[pl.ds(stripe, ROWS_PER_TILE)],
                    out_hbm.at[cid, pl.ds(stripe, ROWS_PER_TILE)])


_sc_agg = functools.partial(
    pl.kernel,
    out_type=jax.ShapeDtypeStruct((NC, PAD_NODES, D), jnp.float32),
    mesh=plsc.VectorSubcoreMesh(
        core_axis_name="c", subcore_axis_name="s",
        num_cores=NC, num_subcores=NS),
    scratch_types=[
        pltpu.VMEM((CHUNK,), jnp.int32),
        pltpu.VMEM((CHUNK,), jnp.int32),
        pltpu.VMEM((CHUNK, D), jnp.float32),
        pltpu.VMEM((ZROWS, D), jnp.float32),
        pltpu.VMEM_SHARED((PAD_NODES, D), jnp.float32),
        pltpu.SemaphoreType.DMA,
    ],
)(_sc_body)


def kernel(feature, norm, edge_index, W, b):
    src = edge_index[0].astype(jnp.int32)
    dst = edge_index[1].astype(jnp.int32)

    h = pl.pallas_call(
        _h_body,
        grid=(_GRID,),
        in_specs=[
            pl.BlockSpec((_BLK, D), lambda i: (i, 0)),
            pl.BlockSpec((_BLK, 1), lambda i: (i, 0)),
        ],
        out_specs=pl.BlockSpec((_BLK, D), lambda i: (i, 0)),
        out_shape=jax.ShapeDtypeStruct((N_NODES, D), jnp.float32),
    )(feature, norm)

    partials = _sc_agg(src, dst, h)

    out = pl.pallas_call(
        _final_body,
        grid=(_GRID,),
        in_specs=[
            pl.BlockSpec((NC, _BLK, D), lambda i: (0, i, 0)),
            pl.BlockSpec((_BLK, 1), lambda i: (i, 0)),
            pl.BlockSpec((D, D), lambda i: (0, 0)),
            pl.BlockSpec((1, D), lambda i: (0, 0)),
        ],
        out_specs=pl.BlockSpec((_BLK, D), lambda i: (i, 0)),
        out_shape=jax.ShapeDtypeStruct((N_NODES, D), jnp.float32),
    )(partials, norm, W, b.reshape(1, D))

    return out
